# baseline (speedup 1.0000x reference)
import functools

import jax
import jax.numpy as jnp
from jax import lax
from jax.experimental import pallas as pl
from jax.experimental.pallas import tpu as pltpu
from jax.experimental.pallas import tpu_sc as plsc


def _proj_body(z_ref, w_ref, b_ref, ze_ref):
    zb = z_ref[0]
    ze = lax.dot_general(zb, w_ref[...], (((0,), (1,)), ((), ())),
                         preferred_element_type=jnp.float32)
    ze_ref[0] = ze + b_ref[...]


def _run_proj(z3, proj_w, proj_b2):
    b, c, hw = z3.shape
    e_dim = proj_w.shape[0]
    return pl.pallas_call(
        _proj_body,
        grid=(b,),
        in_specs=[
            pl.BlockSpec((1, c, hw), lambda i: (i, 0, 0)),
            pl.BlockSpec((e_dim, c), lambda i: (0, 0)),
            pl.BlockSpec((1, e_dim), lambda i: (0, 0)),
        ],
        out_specs=pl.BlockSpec((1, hw, e_dim), lambda i: (i, 0, 0)),
        out_shape=jax.ShapeDtypeStruct((b, hw, e_dim), jnp.float32),
    )(z3, proj_w, proj_b2)


def _dist_body(kc, n_embed, ze_ref, fn_ref, e_ref, ind_ref):
    hw = ze_ref.shape[1]
    ze = ze_ref[0]
    lhs = (2.0 * ze).astype(jnp.bfloat16)
    fn = fn_ref[0]
    half = n_embed // 2
    winners = []
    for h in range(2):
        bv = None
        bi = None
        for c in range(half // kc):
            j0 = h * half + c * kc
            ewc = e_ref[pl.ds(j0, kc), :]
            cn = jnp.sum(ewc * ewc, axis=1)
            s2 = lax.dot_general(lhs, ewc, (((1,), (1,)), ((), ())),
                                 preferred_element_type=jnp.float32)
            v = -((fn - s2) + cn[None, :])
            mv = jnp.max(v, axis=1, keepdims=True)
            ii = lax.broadcasted_iota(jnp.int32, (hw, kc), 1) + j0
            mi = jnp.min(jnp.where(v == mv, ii, n_embed), axis=1, keepdims=True)
            if bv is None:
                bv, bi = mv, mi
            else:
                upd = mv > bv
                bi = jnp.where(upd, mi, bi)
                bv = jnp.where(upd, mv, bv)
        winners.append((bv, bi))
    (v1, i1), (v2, i2) = winners
    take2 = v2 > v1.astype(jnp.bfloat16).astype(jnp.float32)
    ind_ref[0] = jnp.where(take2, i2, i1)


def _run_dist(ze, fn3, embed_w, kc=2048):
    b, hw, e_dim = ze.shape
    n_embed = embed_w.shape[0]
    body = functools.partial(_dist_body, kc, n_embed)
    return pl.pallas_call(
        body,
        grid=(b,),
        in_specs=[
            pl.BlockSpec((1, hw, e_dim), lambda i: (i, 0, 0)),
            pl.BlockSpec((1, hw, 1), lambda i: (i, 0, 0)),
            pl.BlockSpec((n_embed, e_dim), lambda i: (0, 0)),
        ],
        out_specs=pl.BlockSpec((1, hw, 1), lambda i: (i, 0, 0)),
        out_shape=jax.ShapeDtypeStruct((b, hw, 1), jnp.int32),
    )(ze, fn3, embed_w)


_SC_NC = 2
_SC_NS = 16
_SC_CH = 128


def _make_sc_gather(n_tok, e_dim):
    nw = _SC_NC * _SC_NS
    per_w = n_tok // nw
    n_ch = per_w // _SC_CH
    mesh = plsc.VectorSubcoreMesh(core_axis_name="c", subcore_axis_name="s",
                                  num_cores=_SC_NC, num_subcores=_SC_NS)

    @functools.partial(
        pl.kernel,
        mesh=mesh,
        out_type=jax.ShapeDtypeStruct((n_tok, e_dim), jnp.float32),
        scratch_types=[
            pltpu.VMEM((n_ch, _SC_CH), jnp.int32),
            pltpu.VMEM((_SC_CH, e_dim), jnp.float32),
            pltpu.SemaphoreType.DMA,
        ],
    )
    def gather_k(table_hbm, idx_hbm, out_hbm, idx_v, rows_v, sem):
        wid = lax.axis_index("s") * _SC_NC + lax.axis_index("c")
        pltpu.sync_copy(idx_hbm.at[wid], idx_v)
        for j in range(n_ch):
            pltpu.async_copy(table_hbm.at[idx_v.at[j]], rows_v, sem).wait()
            pltpu.sync_copy(rows_v,
                            out_hbm.at[pl.ds(wid * per_w + j * _SC_CH, _SC_CH)])

    return gather_k


def _st_body(ze_ref, zq_ref, out_ref, sum_ref):
    i = pl.program_id(0)
    ze = ze_ref[0]
    zq = zq_ref[0]
    dd = zq - ze
    out_ref[0] = (ze + dd).T
    ps = jnp.sum(dd * dd, keepdims=True)

    @pl.when(i == 0)
    def _():
        sum_ref[...] = ps

    @pl.when(i > 0)
    def _():
        sum_ref[...] += ps


def _run_st(ze, zq):
    b, hw, e_dim = ze.shape
    return pl.pallas_call(
        _st_body,
        grid=(b,),
        in_specs=[
            pl.BlockSpec((1, hw, e_dim), lambda i: (i, 0, 0)),
            pl.BlockSpec((1, hw, e_dim), lambda i: (i, 0, 0)),
        ],
        out_specs=[
            pl.BlockSpec((1, e_dim, hw), lambda i: (i, 0, 0)),
            pl.BlockSpec((1, 1), lambda i: (0, 0)),
        ],
        out_shape=[
            jax.ShapeDtypeStruct((b, e_dim, hw), jnp.float32),
            jax.ShapeDtypeStruct((1, 1), jnp.float32),
        ],
    )(ze, zq)


def kernel(z, proj_w, proj_b, embed_w):
    b, c, h, w = z.shape
    hw = h * w
    e_dim = proj_w.shape[0]
    n_tok = b * hw

    ze = _run_proj(z.reshape(b, c, hw), proj_w, proj_b.reshape(1, e_dim))
    fn = jnp.sum(ze.reshape(n_tok, e_dim) ** 2, axis=1)
    ind3 = _run_dist(ze, fn.reshape(b, hw, 1), embed_w)

    idx = ind3.reshape(_SC_NC * _SC_NS, -1, _SC_CH)
    zq_flat = _make_sc_gather(n_tok, e_dim)(embed_w, idx)
    zq = zq_flat.reshape(b, hw, e_dim)

    zqt, ssum = _run_st(ze, zq)
    m = ssum[0, 0] / jnp.float32(n_tok * e_dim)
    diff = (0.25 * m + m) * 10.0
    return (zqt.reshape(b, e_dim, h, w), diff, ind3.reshape(b, h, w))


# speedup vs baseline: 1.0738x; 1.0738x over previous
import functools

import jax
import jax.numpy as jnp
from jax import lax
from jax.experimental import pallas as pl
from jax.experimental.pallas import tpu as pltpu
from jax.experimental.pallas import tpu_sc as plsc


def _proj_body(z_ref, w_ref, b_ref, ze_ref):
    zb = z_ref[0]
    ze = lax.dot_general(zb, w_ref[...], (((0,), (1,)), ((), ())),
                         preferred_element_type=jnp.float32)
    ze_ref[0] = ze + b_ref[...]


def _run_proj(z3, proj_w, proj_b2):
    b, c, hw = z3.shape
    e_dim = proj_w.shape[0]
    return pl.pallas_call(
        _proj_body,
        grid=(b,),
        in_specs=[
            pl.BlockSpec((1, c, hw), lambda i: (i, 0, 0)),
            pl.BlockSpec((e_dim, c), lambda i: (0, 0)),
            pl.BlockSpec((1, e_dim), lambda i: (0, 0)),
        ],
        out_specs=pl.BlockSpec((1, hw, e_dim), lambda i: (i, 0, 0)),
        out_shape=jax.ShapeDtypeStruct((b, hw, e_dim), jnp.float32),
    )(z3, proj_w, proj_b2)


def _dist_body(kc, n_embed, ze_ref, fn_ref, e_ref, ind_ref):
    hw = ze_ref.shape[1]
    ze = ze_ref[0]
    lhs = (2.0 * ze).astype(jnp.bfloat16)
    fn = fn_ref[0]
    half = n_embed // 2
    winners = []
    for h in range(2):
        bv = None
        bi = None
        for c in range(half // kc):
            j0 = h * half + c * kc
            ewc = e_ref[pl.ds(j0, kc), :]
            cn = jnp.sum(ewc * ewc, axis=1)
            s2 = lax.dot_general(lhs, ewc, (((1,), (1,)), ((), ())),
                                 preferred_element_type=jnp.float32)
            d = (fn - s2) + cn[None, :]
            mv = jnp.min(d, axis=1, keepdims=True)
            ii = lax.broadcasted_iota(jnp.int32, (hw, kc), 1) + j0
            mi = jnp.min(jnp.where(d == mv, ii, n_embed), axis=1, keepdims=True)
            if bv is None:
                bv, bi = mv, mi
            else:
                upd = mv < bv
                bi = jnp.where(upd, mi, bi)
                bv = jnp.where(upd, mv, bv)
        winners.append((bv, bi))
    (d1, i1), (d2, i2) = winners
    take2 = d2 < d1.astype(jnp.bfloat16).astype(jnp.float32)
    ind_ref[0] = jnp.where(take2, i2, i1)


def _run_dist(ze, fn3, embed_w, kc=2048):
    b, hw, e_dim = ze.shape
    n_embed = embed_w.shape[0]
    body = functools.partial(_dist_body, kc, n_embed)
    return pl.pallas_call(
        body,
        grid=(b,),
        in_specs=[
            pl.BlockSpec((1, hw, e_dim), lambda i: (i, 0, 0)),
            pl.BlockSpec((1, hw, 1), lambda i: (i, 0, 0)),
            pl.BlockSpec((n_embed, e_dim), lambda i: (0, 0)),
        ],
        out_specs=pl.BlockSpec((1, hw, 1), lambda i: (i, 0, 0)),
        out_shape=jax.ShapeDtypeStruct((b, hw, 1), jnp.int32),
    )(ze, fn3, embed_w)


_SC_NC = 2
_SC_NS = 16
_SC_CH = 128


def _make_sc_gather(n_tok, e_dim):
    nw = _SC_NC * _SC_NS
    per_w = n_tok // nw
    n_ch = per_w // _SC_CH
    mesh = plsc.VectorSubcoreMesh(core_axis_name="c", subcore_axis_name="s",
                                  num_cores=_SC_NC, num_subcores=_SC_NS)

    @functools.partial(
        pl.kernel,
        mesh=mesh,
        out_type=jax.ShapeDtypeStruct((n_tok, e_dim), jnp.float32),
        scratch_types=[
            pltpu.VMEM((n_ch, _SC_CH), jnp.int32),
            pltpu.VMEM((_SC_CH, e_dim), jnp.float32),
            pltpu.SemaphoreType.DMA,
        ],
    )
    def gather_k(table_hbm, idx_hbm, out_hbm, idx_v, rows_v, sem):
        wid = lax.axis_index("s") * _SC_NC + lax.axis_index("c")
        pltpu.sync_copy(idx_hbm.at[wid], idx_v)
        for j in range(n_ch):
            pltpu.async_copy(table_hbm.at[idx_v.at[j]], rows_v, sem).wait()
            pltpu.sync_copy(rows_v,
                            out_hbm.at[pl.ds(wid * per_w + j * _SC_CH, _SC_CH)])

    return gather_k


def _st_body(ze_ref, zq_ref, out_ref, sum_ref):
    i = pl.program_id(0)
    ze = ze_ref[0]
    zq = zq_ref[0]
    dd = zq - ze
    out_ref[0] = (ze + dd).T
    ps = jnp.sum(dd * dd, keepdims=True)

    @pl.when(i == 0)
    def _():
        sum_ref[...] = ps

    @pl.when(i > 0)
    def _():
        sum_ref[...] += ps


def _run_st(ze, zq):
    b, hw, e_dim = ze.shape
    return pl.pallas_call(
        _st_body,
        grid=(b,),
        in_specs=[
            pl.BlockSpec((1, hw, e_dim), lambda i: (i, 0, 0)),
            pl.BlockSpec((1, hw, e_dim), lambda i: (i, 0, 0)),
        ],
        out_specs=[
            pl.BlockSpec((1, e_dim, hw), lambda i: (i, 0, 0)),
            pl.BlockSpec((1, 1), lambda i: (0, 0)),
        ],
        out_shape=[
            jax.ShapeDtypeStruct((b, e_dim, hw), jnp.float32),
            jax.ShapeDtypeStruct((1, 1), jnp.float32),
        ],
    )(ze, zq)


def kernel(z, proj_w, proj_b, embed_w):
    b, c, h, w = z.shape
    hw = h * w
    e_dim = proj_w.shape[0]
    n_tok = b * hw

    ze = _run_proj(z.reshape(b, c, hw), proj_w, proj_b.reshape(1, e_dim))
    fn = jnp.sum(ze.reshape(n_tok, e_dim) ** 2, axis=1)
    ind3 = _run_dist(ze, fn.reshape(b, hw, 1), embed_w)

    idx = ind3.reshape(_SC_NC * _SC_NS, -1, _SC_CH)
    zq_flat = _make_sc_gather(n_tok, e_dim)(embed_w, idx)
    zq = zq_flat.reshape(b, hw, e_dim)

    zqt, ssum = _run_st(ze, zq)
    m = ssum[0, 0] / jnp.float32(n_tok * e_dim)
    diff = (0.25 * m + m) * 10.0
    return (zqt.reshape(b, e_dim, h, w), diff, ind3.reshape(b, h, w))


# speedup vs baseline: 1.0923x; 1.0172x over previous
import functools

import jax
import jax.numpy as jnp
from jax import lax
from jax.experimental import pallas as pl
from jax.experimental.pallas import tpu as pltpu
from jax.experimental.pallas import tpu_sc as plsc


def _proj_body(z_ref, w_ref, b_ref, ze_ref):
    zb = z_ref[0]
    ze = lax.dot_general(zb, w_ref[...], (((0,), (1,)), ((), ())),
                         preferred_element_type=jnp.float32)
    ze_ref[0] = ze + b_ref[...]


def _run_proj(z3, proj_w, proj_b2):
    b, c, hw = z3.shape
    e_dim = proj_w.shape[0]
    return pl.pallas_call(
        _proj_body,
        grid=(b,),
        in_specs=[
            pl.BlockSpec((1, c, hw), lambda i: (i, 0, 0)),
            pl.BlockSpec((e_dim, c), lambda i: (0, 0)),
            pl.BlockSpec((1, e_dim), lambda i: (0, 0)),
        ],
        out_specs=pl.BlockSpec((1, hw, e_dim), lambda i: (i, 0, 0)),
        out_shape=jax.ShapeDtypeStruct((b, hw, e_dim), jnp.float32),
    )(z3, proj_w, proj_b2)


def _dist_body(kc, n_embed, ze_ref, fn_ref, e_ref, ind_ref):
    hw = ze_ref.shape[1]
    ze = ze_ref[0]
    lhs = (2.0 * ze).astype(jnp.bfloat16)
    fn = fn_ref[0]
    half = n_embed // 2
    winners = []
    for h in range(2):
        bv = None
        bi = None
        for c in range(half // kc):
            j0 = h * half + c * kc
            ewc = e_ref[pl.ds(j0, kc), :]
            cn = jnp.sum(ewc * ewc, axis=1)
            s2 = lax.dot_general(lhs, ewc, (((1,), (1,)), ((), ())),
                                 preferred_element_type=jnp.float32)
            d = (fn - s2) + cn[None, :]
            mv = jnp.min(d, axis=1, keepdims=True)
            ii = lax.broadcasted_iota(jnp.int32, (hw, kc), 1) + j0
            mi = jnp.min(jnp.where(d == mv, ii, n_embed), axis=1, keepdims=True)
            if bv is None:
                bv, bi = mv, mi
            else:
                upd = mv < bv
                bi = jnp.where(upd, mi, bi)
                bv = jnp.where(upd, mv, bv)
        winners.append((bv, bi))
    (d1, i1), (d2, i2) = winners
    take2 = d2 < d1.astype(jnp.bfloat16).astype(jnp.float32)
    ind_ref[0] = jnp.where(take2, i2, i1)


def _run_dist(ze, fn3, embed_w, kc=4096):
    b, hw, e_dim = ze.shape
    n_embed = embed_w.shape[0]
    body = functools.partial(_dist_body, kc, n_embed)
    return pl.pallas_call(
        body,
        grid=(b,),
        in_specs=[
            pl.BlockSpec((1, hw, e_dim), lambda i: (i, 0, 0)),
            pl.BlockSpec((1, hw, 1), lambda i: (i, 0, 0)),
            pl.BlockSpec((n_embed, e_dim), lambda i: (0, 0)),
        ],
        out_specs=pl.BlockSpec((1, hw, 1), lambda i: (i, 0, 0)),
        out_shape=jax.ShapeDtypeStruct((b, hw, 1), jnp.int32),
    )(ze, fn3, embed_w)


_SC_NC = 2
_SC_NS = 16
_SC_CH = 128


def _make_sc_gather(n_tok, e_dim):
    nw = _SC_NC * _SC_NS
    per_w = n_tok // nw
    n_ch = per_w // _SC_CH
    mesh = plsc.VectorSubcoreMesh(core_axis_name="c", subcore_axis_name="s",
                                  num_cores=_SC_NC, num_subcores=_SC_NS)

    @functools.partial(
        pl.kernel,
        mesh=mesh,
        out_type=jax.ShapeDtypeStruct((n_tok, e_dim), jnp.float32),
        scratch_types=[
            pltpu.VMEM((n_ch, _SC_CH), jnp.int32),
            pltpu.VMEM((_SC_CH, e_dim), jnp.float32),
            pltpu.SemaphoreType.DMA,
        ],
    )
    def gather_k(table_hbm, idx_hbm, out_hbm, idx_v, rows_v, sem):
        wid = lax.axis_index("s") * _SC_NC + lax.axis_index("c")
        pltpu.sync_copy(idx_hbm.at[wid], idx_v)
        for j in range(n_ch):
            pltpu.async_copy(table_hbm.at[idx_v.at[j]], rows_v, sem).wait()
            pltpu.sync_copy(rows_v,
                            out_hbm.at[pl.ds(wid * per_w + j * _SC_CH, _SC_CH)])

    return gather_k


def _st_body(ze_ref, zq_ref, out_ref, sum_ref):
    i = pl.program_id(0)
    ze = ze_ref[0]
    zq = zq_ref[0]
    dd = zq - ze
    out_ref[0] = (ze + dd).T
    ps = jnp.sum(dd * dd, keepdims=True)

    @pl.when(i == 0)
    def _():
        sum_ref[...] = ps

    @pl.when(i > 0)
    def _():
        sum_ref[...] += ps


def _run_st(ze, zq):
    b, hw, e_dim = ze.shape
    return pl.pallas_call(
        _st_body,
        grid=(b,),
        in_specs=[
            pl.BlockSpec((1, hw, e_dim), lambda i: (i, 0, 0)),
            pl.BlockSpec((1, hw, e_dim), lambda i: (i, 0, 0)),
        ],
        out_specs=[
            pl.BlockSpec((1, e_dim, hw), lambda i: (i, 0, 0)),
            pl.BlockSpec((1, 1), lambda i: (0, 0)),
        ],
        out_shape=[
            jax.ShapeDtypeStruct((b, e_dim, hw), jnp.float32),
            jax.ShapeDtypeStruct((1, 1), jnp.float32),
        ],
    )(ze, zq)


def kernel(z, proj_w, proj_b, embed_w):
    b, c, h, w = z.shape
    hw = h * w
    e_dim = proj_w.shape[0]
    n_tok = b * hw

    ze = _run_proj(z.reshape(b, c, hw), proj_w, proj_b.reshape(1, e_dim))
    fn = jnp.sum(ze.reshape(n_tok, e_dim) ** 2, axis=1)
    ind3 = _run_dist(ze, fn.reshape(b, hw, 1), embed_w)

    idx = ind3.reshape(_SC_NC * _SC_NS, -1, _SC_CH)
    zq_flat = _make_sc_gather(n_tok, e_dim)(embed_w, idx)
    zq = zq_flat.reshape(b, hw, e_dim)

    zqt, ssum = _run_st(ze, zq)
    m = ssum[0, 0] / jnp.float32(n_tok * e_dim)
    diff = (0.25 * m + m) * 10.0
    return (zqt.reshape(b, e_dim, h, w), diff, ind3.reshape(b, h, w))
